# Initial kernel scaffold; baseline (speedup 1.0000x reference)
import functools

import jax
import jax.numpy as jnp
from jax import lax
from jax.experimental import pallas as pl
from jax.experimental.pallas import tpu as pltpu
from jax.experimental.pallas import tpu_sc as plsc

VOCAB = 100000
HIDDEN = 64
BATCH = 4096
SEQ = 50
LETTERS = 20
SEQ_FLAT = SEQ * LETTERS
N_CHUNKS = 8
CHUNK = SEQ_FLAT // N_CHUNKS

_info = plsc.get_sparse_core_info()
NC, NS = _info.num_cores, _info.num_subcores
NW = NC * NS
B_PER_W = BATCH // NW


def _sc_body(ids_hbm, table_hbm, pos_hbm, out_hbm, idx_v, rows_v, pos_v, out_v, sem):
    wid = lax.axis_index("s") * NC + lax.axis_index("c")
    pltpu.sync_copy(pos_hbm.at[pl.ds(0, SEQ)], pos_v)

    def per_row(i, carry):
        b = wid * B_PER_W + i
        pltpu.sync_copy(ids_hbm.at[b], idx_v)
        copies = [
            pltpu.async_copy(
                table_hbm.at[idx_v.at[j]],
                rows_v.at[pl.ds(j * CHUNK, CHUNK)],
                sem,
            )
            for j in range(N_CHUNKS)
        ]
        for c in copies:
            c.wait()

        def per_seg(s, carry2):
            for h in range(HIDDEN // 16):
                acc = pos_v[s, pl.ds(h * 16, 16)]
                for l in range(LETTERS):
                    acc = acc + rows_v[s * LETTERS + l, pl.ds(h * 16, 16)]
                out_v[s, pl.ds(h * 16, 16)] = acc
            return carry2

        lax.fori_loop(0, SEQ, per_seg, 0)
        pltpu.sync_copy(out_v, out_hbm.at[b])
        return carry

    lax.fori_loop(0, B_PER_W, per_row, 0)


_sc_kernel = functools.partial(
    pl.kernel,
    out_type=jax.ShapeDtypeStruct((BATCH, SEQ, HIDDEN), jnp.float32),
    mesh=plsc.VectorSubcoreMesh(core_axis_name="c", subcore_axis_name="s"),
    scratch_types=[
        pltpu.VMEM((N_CHUNKS, CHUNK), jnp.int32),
        pltpu.VMEM((SEQ_FLAT, HIDDEN), jnp.float32),
        pltpu.VMEM((SEQ, HIDDEN), jnp.float32),
        pltpu.VMEM((SEQ, HIDDEN), jnp.float32),
        pltpu.SemaphoreType.DMA,
    ],
)(_sc_body)


@jax.jit
def kernel(input_ids, tri_table, pos_table):
    ids3 = input_ids.reshape(BATCH, N_CHUNKS, CHUNK)
    return _sc_kernel(ids3, tri_table, pos_table)


# speedup vs baseline: 13.1710x; 13.1710x over previous
import functools

import jax
import jax.numpy as jnp
from jax import lax
from jax.experimental import pallas as pl
from jax.experimental.pallas import tpu as pltpu
from jax.experimental.pallas import tpu_sc as plsc

VOCAB = 100000
HIDDEN = 64
BATCH = 4096
SEQ = 50
LETTERS = 20
SEQ_FLAT = SEQ * LETTERS
N_CHUNKS = 8
CHUNK = SEQ_FLAT // N_CHUNKS
POS_STAGE = 56

_info = plsc.get_sparse_core_info()
NC, NS = _info.num_cores, _info.num_subcores
NW = NC * NS
B_PER_W = BATCH // NW


def _sc_body(ids_hbm, table_hbm, pos_hbm, out_hbm, idx_v, rows_v, pos_v, out_v, sem):
    wid = lax.axis_index("s") * NC + lax.axis_index("c")
    pltpu.sync_copy(pos_hbm.at[pl.ds(0, POS_STAGE)], pos_v)

    def per_row(i, carry):
        b = wid * B_PER_W + i
        pltpu.sync_copy(ids_hbm.at[b], idx_v)
        copies = [
            pltpu.async_copy(
                table_hbm.at[idx_v.at[j]],
                rows_v.at[pl.ds(j * CHUNK, CHUNK)],
                sem,
            )
            for j in range(N_CHUNKS)
        ]
        for c in copies:
            c.wait()

        def per_seg(s, carry2):
            for h in range(HIDDEN // 16):
                acc = pos_v[s, pl.ds(h * 16, 16)]
                for l in range(LETTERS):
                    acc = acc + rows_v[s * LETTERS + l, pl.ds(h * 16, 16)]
                out_v[s, pl.ds(h * 16, 16)] = acc
            return carry2

        lax.fori_loop(0, SEQ, per_seg, 0)
        pltpu.sync_copy(out_v, out_hbm.at[b])
        return carry

    lax.fori_loop(0, B_PER_W, per_row, 0)


_sc_kernel = functools.partial(
    pl.kernel,
    out_type=jax.ShapeDtypeStruct((BATCH, SEQ, HIDDEN), jnp.float32),
    mesh=plsc.VectorSubcoreMesh(core_axis_name="c", subcore_axis_name="s"),
    scratch_types=[
        pltpu.VMEM((N_CHUNKS, CHUNK), jnp.int32),
        pltpu.VMEM((SEQ_FLAT, HIDDEN), jnp.float32),
        pltpu.VMEM((POS_STAGE, HIDDEN), jnp.float32),
        pltpu.VMEM((SEQ, HIDDEN), jnp.float32),
        pltpu.SemaphoreType.DMA,
    ],
    compiler_params=pltpu.CompilerParams(use_tc_tiling_on_sc=False),
)(_sc_body)


@jax.jit
def kernel(input_ids, tri_table, pos_table):
    ids3 = input_ids.reshape(BATCH, N_CHUNKS, CHUNK)
    return _sc_kernel(ids3, tri_table, pos_table)


# speedup vs baseline: 18.0739x; 1.3723x over previous
import functools

import jax
import jax.numpy as jnp
from jax import lax
from jax.experimental import pallas as pl
from jax.experimental.pallas import tpu as pltpu
from jax.experimental.pallas import tpu_sc as plsc

VOCAB = 100000
HIDDEN = 64
BATCH = 4096
SEQ = 50
LETTERS = 20
SEQ_FLAT = SEQ * LETTERS
CHUNK = 125
CHUNKS_PER_HALF = 4
HALF = CHUNK * CHUNKS_PER_HALF
SEGS_PER_HALF = HALF // LETTERS
POS_STAGE = 56
NVREG = HIDDEN // 16

_info = plsc.get_sparse_core_info()
NC, NS = _info.num_cores, _info.num_subcores
NW = NC * NS
B_PER_W = BATCH // NW
NG = 2 * B_PER_W


def _sc_body(
    ids_hbm,
    table_hbm,
    pos_hbm,
    out_hbm,
    idx_v,
    rows_v,
    pos_v,
    out_v,
    sem_g0,
    sem_g1,
    sem_i0,
    sem_i1,
    sem_o0,
    sem_o1,
):
    wid = lax.axis_index("s") * NC + lax.axis_index("c")
    pltpu.sync_copy(pos_hbm.at[pl.ds(0, POS_STAGE)], pos_v)

    sem_g = [sem_g0, sem_g1]
    sem_i = [sem_i0, sem_i1]
    sem_o = [sem_o0, sem_o1]

    def fire_idx(g, parity):
        b = wid * B_PER_W + g // 2
        ib = g - (g // 4) * 4
        pltpu.async_copy(
            ids_hbm.at[b, pl.ds(parity * CHUNKS_PER_HALF, CHUNKS_PER_HALF)],
            idx_v.at[pl.ds(ib * CHUNKS_PER_HALF, CHUNKS_PER_HALF)],
            sem_i[parity],
        )

    def wait_idx(parity):
        pltpu.make_async_copy(
            ids_hbm.at[0, pl.ds(0, CHUNKS_PER_HALF)],
            idx_v.at[pl.ds(0, CHUNKS_PER_HALF)],
            sem_i[parity],
        ).wait()

    def fire_gathers(g, parity):
        ib = g - (g // 4) * 4
        for j in range(CHUNKS_PER_HALF):
            pltpu.async_copy(
                table_hbm.at[idx_v.at[ib * CHUNKS_PER_HALF + j]],
                rows_v.at[pl.ds(parity * HALF + j * CHUNK, CHUNK)],
                sem_g[parity],
            )

    def wait_gathers(parity):
        pltpu.make_async_copy(
            table_hbm.at[pl.ds(0, HALF)],
            rows_v.at[pl.ds(0, HALF)],
            sem_g[parity],
        ).wait()

    def fire_out(g, parity):
        b = wid * B_PER_W + g // 2
        pltpu.async_copy(
            out_v.at[pl.ds(parity * SEGS_PER_HALF, SEGS_PER_HALF)],
            out_hbm.at[b, pl.ds(parity * SEGS_PER_HALF, SEGS_PER_HALF)],
            sem_o[parity],
        )

    def wait_out(parity):
        pltpu.make_async_copy(
            out_v.at[pl.ds(0, SEGS_PER_HALF)],
            out_hbm.at[0, pl.ds(0, SEGS_PER_HALF)],
            sem_o[parity],
        ).wait()

    def reduce_half(parity):
        row_base = parity * HALF
        out_base = parity * SEGS_PER_HALF
        seg_base = parity * SEGS_PER_HALF

        def per_seg(so, carry):
            for si in range(5):
                s_local = so * 5 + si
                r0 = row_base + s_local * LETTERS
                for h in range(NVREG):
                    acc = pos_v[seg_base + s_local, pl.ds(h * 16, 16)]
                    for l in range(LETTERS):
                        acc = acc + rows_v[r0 + l, pl.ds(h * 16, 16)]
                    out_v[out_base + s_local, pl.ds(h * 16, 16)] = acc
            return carry

        lax.fori_loop(0, SEGS_PER_HALF // 5, per_seg, 0)

    fire_idx(0, 0)
    fire_idx(1, 1)
    wait_idx(0)
    fire_gathers(0, 0)

    def half_step(g, parity):
        @pl.when(g + 2 < NG)
        def _():
            fire_idx(g + 2, parity)

        @pl.when(g + 1 < NG)
        def _():
            wait_idx(1 - parity)
            fire_gathers(g + 1, 1 - parity)

        wait_gathers(parity)

        @pl.when(g >= 2)
        def _():
            wait_out(parity)

        reduce_half(parity)
        fire_out(g, parity)

    def step(i, carry):
        half_step(2 * i, 0)
        half_step(2 * i + 1, 1)
        return carry

    lax.fori_loop(0, B_PER_W, step, 0)
    wait_out(0)
    wait_out(1)


_sc_kernel = functools.partial(
    pl.kernel,
    out_type=jax.ShapeDtypeStruct((BATCH, SEQ, HIDDEN), jnp.float32),
    mesh=plsc.VectorSubcoreMesh(core_axis_name="c", subcore_axis_name="s"),
    scratch_types=[
        pltpu.VMEM((4 * CHUNKS_PER_HALF, CHUNK), jnp.int32),
        pltpu.VMEM((2 * HALF, HIDDEN), jnp.float32),
        pltpu.VMEM((POS_STAGE, HIDDEN), jnp.float32),
        pltpu.VMEM((2 * SEGS_PER_HALF, HIDDEN), jnp.float32),
        pltpu.SemaphoreType.DMA,
        pltpu.SemaphoreType.DMA,
        pltpu.SemaphoreType.DMA,
        pltpu.SemaphoreType.DMA,
        pltpu.SemaphoreType.DMA,
        pltpu.SemaphoreType.DMA,
    ],
    compiler_params=pltpu.CompilerParams(use_tc_tiling_on_sc=False),
)(_sc_body)


@jax.jit
def kernel(input_ids, tri_table, pos_table):
    ids3 = input_ids.reshape(BATCH, 2 * CHUNKS_PER_HALF, CHUNK)
    return _sc_kernel(ids3, tri_table, pos_table)


# speedup vs baseline: 31.3506x; 1.7346x over previous
import functools

import jax
import jax.numpy as jnp
from jax import lax
from jax.experimental import pallas as pl
from jax.experimental.pallas import tpu as pltpu
from jax.experimental.pallas import tpu_sc as plsc

VOCAB = 100000
HIDDEN = 64
BATCH = 4096
SEQ = 50
LETTERS = 20
SEQ_FLAT = SEQ * LETTERS
CHUNK = 125
CHUNKS_PER_HALF = 4
HALF = CHUNK * CHUNKS_PER_HALF
SEGS_PER_HALF = HALF // LETTERS
POS_STAGE = 56
NBLK = HIDDEN // 32

_info = plsc.get_sparse_core_info()
NC, NS = _info.num_cores, _info.num_subcores
NW = NC * NS
B_PER_W = BATCH // NW
NG = 2 * B_PER_W


def _sc_body(
    ids_hbm,
    table_hbm,
    pos_hbm,
    out_hbm,
    idx_v,
    rows_v,
    pos_v,
    out_v,
    sem_g0,
    sem_g1,
    sem_i0,
    sem_i1,
    sem_o0,
    sem_o1,
):
    wid = lax.axis_index("s") * NC + lax.axis_index("c")
    pltpu.sync_copy(pos_hbm.at[pl.ds(0, POS_STAGE)], pos_v)

    sem_g = [sem_g0, sem_g1]
    sem_i = [sem_i0, sem_i1]
    sem_o = [sem_o0, sem_o1]

    def fire_idx(g, parity):
        b = wid * B_PER_W + g // 2
        ib = g - (g // 4) * 4
        pltpu.async_copy(
            ids_hbm.at[b, pl.ds(parity * CHUNKS_PER_HALF, CHUNKS_PER_HALF)],
            idx_v.at[pl.ds(ib * CHUNKS_PER_HALF, CHUNKS_PER_HALF)],
            sem_i[parity],
        )

    def wait_idx(parity):
        pltpu.make_async_copy(
            ids_hbm.at[0, pl.ds(0, CHUNKS_PER_HALF)],
            idx_v.at[pl.ds(0, CHUNKS_PER_HALF)],
            sem_i[parity],
        ).wait()

    def fire_gathers(g, parity):
        ib = g - (g // 4) * 4
        for j in range(CHUNKS_PER_HALF):
            pltpu.async_copy(
                table_hbm.at[idx_v.at[ib * CHUNKS_PER_HALF + j]],
                rows_v.at[pl.ds(parity * HALF + j * CHUNK, CHUNK)],
                sem_g[parity],
            )

    def wait_gathers(parity):
        pltpu.make_async_copy(
            table_hbm.at[pl.ds(0, HALF)],
            rows_v.at[pl.ds(0, HALF)],
            sem_g[parity],
        ).wait()

    def fire_out(g, parity):
        b = wid * B_PER_W + g // 2
        pltpu.async_copy(
            out_v.at[pl.ds(parity * SEGS_PER_HALF * HIDDEN, SEGS_PER_HALF * HIDDEN)],
            out_hbm.at[b, pl.ds(parity * SEGS_PER_HALF * HIDDEN, SEGS_PER_HALF * HIDDEN)],
            sem_o[parity],
        )

    def wait_out(parity):
        pltpu.make_async_copy(
            out_v.at[pl.ds(0, SEGS_PER_HALF * HIDDEN)],
            out_hbm.at[0, pl.ds(0, SEGS_PER_HALF * HIDDEN)],
            sem_o[parity],
        ).wait()

    def reduce_half(parity):
        row_base = parity * HALF
        out_base = parity * SEGS_PER_HALF
        ve = lax.iota(jnp.int32, 16) * 2

        def per_seg(so, carry):
            for si in range(5):
                s_local = so * 5 + si
                r0 = row_base + s_local * LETTERS
                for k in range(NBLK):
                    acc_e = pos_v[out_base + s_local, pl.ds(k * 32, 16)]
                    acc_o = pos_v[out_base + s_local, pl.ds(k * 32 + 16, 16)]
                    for l in range(LETTERS):
                        v = rows_v[r0 + l, pl.ds(k * 32, 32)]
                        e, o = plsc.unpack(v, format=plsc.PackFormat.INTERLEAVED)
                        acc_e = acc_e + e
                        acc_o = acc_o + o
                    base = (out_base + s_local) * HIDDEN + 32 * k
                    plsc.store_scatter(out_v, [ve + base], acc_e)
                    plsc.store_scatter(out_v, [ve + (base + 1)], acc_o)
            return carry

        lax.fori_loop(0, SEGS_PER_HALF // 5, per_seg, 0)

    fire_idx(0, 0)
    fire_idx(1, 1)
    wait_idx(0)
    fire_gathers(0, 0)

    def half_step(g, parity):
        @pl.when(g + 2 < NG)
        def _():
            fire_idx(g + 2, parity)

        @pl.when(g + 1 < NG)
        def _():
            wait_idx(1 - parity)
            fire_gathers(g + 1, 1 - parity)

        wait_gathers(parity)

        @pl.when(g >= 2)
        def _():
            wait_out(parity)

        reduce_half(parity)
        fire_out(g, parity)

    def step(i, carry):
        half_step(2 * i, 0)
        half_step(2 * i + 1, 1)
        return carry

    lax.fori_loop(0, B_PER_W, step, 0)
    wait_out(0)
    wait_out(1)


_sc_kernel = functools.partial(
    pl.kernel,
    out_type=jax.ShapeDtypeStruct((BATCH, SEQ * HIDDEN), jnp.float32),
    mesh=plsc.VectorSubcoreMesh(core_axis_name="c", subcore_axis_name="s"),
    scratch_types=[
        pltpu.VMEM((4 * CHUNKS_PER_HALF, CHUNK), jnp.int32),
        pltpu.VMEM((2 * HALF, HIDDEN), jnp.bfloat16),
        pltpu.VMEM((POS_STAGE, HIDDEN), jnp.float32),
        pltpu.VMEM((2 * SEGS_PER_HALF * HIDDEN,), jnp.float32),
        pltpu.SemaphoreType.DMA,
        pltpu.SemaphoreType.DMA,
        pltpu.SemaphoreType.DMA,
        pltpu.SemaphoreType.DMA,
        pltpu.SemaphoreType.DMA,
        pltpu.SemaphoreType.DMA,
    ],
    compiler_params=pltpu.CompilerParams(
        use_tc_tiling_on_sc=False, needs_layout_passes=False
    ),
)(_sc_body)


@jax.jit
def kernel(input_ids, tri_table, pos_table):
    ids3 = input_ids.reshape(BATCH, 2 * CHUNKS_PER_HALF, CHUNK)
    table16 = tri_table.astype(jnp.bfloat16)
    pos_de = (
        pos_table.reshape(-1, NBLK, 16, 2).transpose(0, 1, 3, 2).reshape(-1, HIDDEN)
    )
    out = _sc_kernel(ids3, table16, pos_de)
    return out.reshape(BATCH, SEQ, HIDDEN)
